# baseline (speedup 1.0000x reference)
import functools

import jax
import jax.numpy as jnp
from jax import lax
from jax.experimental import pallas as pl
from jax.experimental.pallas import tpu as pltpu
from jax.experimental.pallas import tpu_sc as plsc

N_ABS = 512
N_MAPS = 100
MAX_POS = 1024
BATCH = 16384

NUM_CORES = 2
NUM_WORKERS = 16 * NUM_CORES
PER_W = BATCH // NUM_WORKERS
ROWS = PER_W // 128
LANES = 16

_E_CHUNK = 64
_E_NCHUNK = PER_W // _E_CHUNK
_E_NB = 3


def _sc_onehot_body(tbl_hbm, m_hbm, p_hbm, out_hbm,
                    m_v, p_v, idx_v, c_v, buf0, buf1, buf2,
                    gsem, dsem0, dsem1, dsem2):
    wid = lax.axis_index("c") * 16 + lax.axis_index("s")
    base = wid * PER_W
    mcp = pltpu.async_copy(m_hbm.at[wid], m_v, gsem)
    pcp = pltpu.async_copy(p_hbm.at[wid], p_v, gsem)
    mcp.wait()
    pcp.wait()
    for j in range(ROWS):
        for i in range(128 // LANES):
            sl = pl.ds(i * LANES, LANES)
            idx_v[j, sl] = m_v[j, sl] * MAX_POS + p_v[j, sl]
    gcopies = [pltpu.async_copy(tbl_hbm.at[idx_v.at[j]], c_v.at[j], gsem)
               for j in range(ROWS)]
    zero16 = jnp.zeros((LANES,), jnp.float32)
    bufs = (buf0, buf1, buf2)
    sems = (dsem0, dsem1, dsem2)
    row_iota = lax.iota(jnp.int32, LANES)
    ones16 = jnp.full((LANES,), 1.0, jnp.float32)

    def _zero_rows(buf, start, nrows):
        def _zrow(r, _):
            for k in range(N_ABS // LANES):
                buf[r, pl.ds(k * LANES, LANES)] = zero16
            return 0
        lax.fori_loop(start, start + nrows, _zrow, 0)

    def _set(buf, buf_row, gi, val):
        cvals = c_v[gi // 128, pl.ds(gi % 128, LANES)]
        plsc.store_scatter(buf, [row_iota + buf_row, cvals], val)

    _zero_rows(buf0, 0, _E_CHUNK)
    for cp in gcopies:
        cp.wait()
    dcopies = [None] * _E_NCHUNK
    for ch in range(_E_NCHUNK):
        b = ch % _E_NB
        if ch >= _E_NB:
            dcopies[ch - _E_NB].wait()
            pch = ch - _E_NB
            for k in range(_E_CHUNK // LANES):
                _set(bufs[b], k * LANES, pch * _E_CHUNK + k * LANES, zero16)
        for k in range(_E_CHUNK // LANES):
            _set(bufs[b], k * LANES, ch * _E_CHUNK + k * LANES, ones16)
        dcopies[ch] = pltpu.async_copy(
            bufs[b], out_hbm.at[pl.ds(base + ch * _E_CHUNK, _E_CHUNK), :], sems[b])
        if ch + 1 < _E_NB:
            _zero_rows(bufs[ch + 1], 0, _E_CHUNK)
    for ch in range(_E_NCHUNK - _E_NB, _E_NCHUNK):
        dcopies[ch].wait()


@functools.cache
def _sc_onehot():
    return pl.kernel(
        _sc_onehot_body,
        out_type=jax.ShapeDtypeStruct((BATCH, N_ABS), jnp.float32),
        mesh=plsc.VectorSubcoreMesh(core_axis_name="c", subcore_axis_name="s",
                                    num_cores=NUM_CORES),
        compiler_params=pltpu.CompilerParams(needs_layout_passes=False),
        scratch_types=[
            pltpu.VMEM((ROWS, 128), jnp.int32),
            pltpu.VMEM((ROWS, 128), jnp.int32),
            pltpu.VMEM((ROWS, 128), jnp.int32),
            pltpu.VMEM((ROWS, 128), jnp.int32),
            pltpu.VMEM((_E_CHUNK, N_ABS), jnp.float32),
            pltpu.VMEM((_E_CHUNK, N_ABS), jnp.float32),
            pltpu.VMEM((_E_CHUNK, N_ABS), jnp.float32),
            pltpu.SemaphoreType.DMA,
            pltpu.SemaphoreType.DMA,
            pltpu.SemaphoreType.DMA,
            pltpu.SemaphoreType.DMA,
        ],
    )


def kernel(map_ids, pos, abs_table):
    m3 = map_ids.astype(jnp.int32).reshape(NUM_WORKERS, ROWS, 128)
    p3 = pos.astype(jnp.int32).reshape(NUM_WORKERS, ROWS, 128)
    tbl = abs_table.astype(jnp.int32).reshape(-1)
    return _sc_onehot()(tbl, m3, p3)
